# Initial kernel scaffold; baseline (speedup 1.0000x reference)
import jax
import jax.numpy as jnp
from jax.experimental import pallas as pl


def kernel(r_ids, w):
    raise NotImplementedError("write your pallas kernel here")


# speedup vs baseline: 122.6228x; 122.6228x over previous
import functools

import jax
import jax.numpy as jnp
from jax import lax
from jax.experimental import pallas as pl
from jax.experimental.pallas import tpu as pltpu
from jax.experimental.pallas import tpu_sc as plsc

B_MAX = 10.0
NUM_REL = 1_000_000

NC, NS, L = 2, 16, 16
NW = NC * NS

ROWS, COLS = 16384, 200
TOTAL = ROWS * COLS
PER_W = TOTAL // NW
CHUNK = 12_800
NCHUNK = PER_W // CHUNK

PER_TILE_TBL = 64_000
TBL_CHUNKS = PER_TILE_TBL // CHUNK
TBL_PAD = NS * PER_TILE_TBL


def _body(w_hbm, ids_hbm, out_hbm, tbl_s, idx_v, val_v, sem):
    cid = lax.axis_index("c")
    sid = lax.axis_index("s")

    t0 = sid * PER_TILE_TBL

    def s1(c, carry):
        off = t0 + c * CHUNK
        pltpu.sync_copy(w_hbm.at[pl.ds(off, CHUNK)], val_v)

        def sig(i, inner):
            x = val_v[pl.ds(i * L, L)]
            val_v[pl.ds(i * L, L)] = B_MAX / (1.0 + jnp.exp(-x))
            return inner

        lax.fori_loop(0, CHUNK // L, sig, 0)
        pltpu.sync_copy(val_v, tbl_s.at[pl.ds(off, CHUNK)])
        return carry

    lax.fori_loop(0, TBL_CHUNKS, s1, 0)
    plsc.subcore_barrier()

    wid = sid * NC + cid
    base = wid * PER_W

    def s2(c, carry):
        off = base + c * CHUNK
        pltpu.sync_copy(ids_hbm.at[pl.ds(off, CHUNK)], idx_v)
        pltpu.async_copy(tbl_s.at[idx_v], val_v, sem).wait()
        pltpu.sync_copy(val_v, out_hbm.at[pl.ds(off, CHUNK)])
        return carry

    lax.fori_loop(0, NCHUNK, s2, 0)


_mesh = plsc.VectorSubcoreMesh(core_axis_name="c", subcore_axis_name="s")

_sc_call = functools.partial(
    pl.kernel,
    out_type=jax.ShapeDtypeStruct((TOTAL,), jnp.float32),
    mesh=_mesh,
    scratch_types=[
        pltpu.VMEM_SHARED((TBL_PAD,), jnp.float32),
        pltpu.VMEM((CHUNK,), jnp.int32),
        pltpu.VMEM((CHUNK,), jnp.float32),
        pltpu.SemaphoreType.DMA,
    ],
)(_body)


def kernel(r_ids, w):
    w_flat = jnp.pad(w[:, 0], (0, TBL_PAD - NUM_REL))
    ids_flat = r_ids.reshape(-1).astype(jnp.int32)
    out = _sc_call(w_flat, ids_flat)
    return out.reshape(ROWS, COLS)
